# baseline (speedup 1.0000x reference)
import functools

import jax
import jax.numpy as jnp
from jax import lax
from jax.experimental import pallas as pl
from jax.experimental.pallas import tpu as pltpu
from jax.experimental.pallas import tpu_sc as plsc

N = 10000
E = 320000
D = 128
NC = 2
NS = 16
CHUNK = 128
NR = 5
LAG = NR - 1
GB = 10
CHUNKS_PER_TILE = 160
NGRP = CHUNKS_PER_TILE // GB
CHUNKS_PAD = NS * CHUNKS_PER_TILE
HALF = 5120
ROWS_PER_TILE = HALF // NS
LAST_ROWS = N - (NC * HALF - ROWS_PER_TILE)


def _mlp_body(x_ref, w1a_ref, w1b_ref, s_ref, b1_ref, w2_ref, b2_ref, o_ref):
    b1eff = b1_ref[...] + jnp.dot(s_ref[...], w1b_ref[...],
                                  preferred_element_type=jnp.float32)
    h = jnp.dot(x_ref[...], w1a_ref[...], preferred_element_type=jnp.float32)
    h = jnp.maximum(h + b1eff, 0.0)
    o = jnp.dot(h, w2_ref[...], preferred_element_type=jnp.float32)
    o_ref[...] = o + b2_ref[...]


def _node_mlp(x, scalars, W1, b1, W2, b2):
    blk = 1000
    grid = N // blk
    full = lambda shape: pl.BlockSpec(shape, lambda i: (0,) * len(shape))
    return pl.pallas_call(
        _mlp_body,
        grid=(grid,),
        in_specs=[
            pl.BlockSpec((blk, D), lambda i: (i, 0)),
            full((D, D)),
            full((4, D)),
            full((1, 4)),
            full((1, D)),
            full((D, D)),
            full((1, D)),
        ],
        out_specs=pl.BlockSpec((blk, D), lambda i: (i, 0)),
        out_shape=jax.ShapeDtypeStruct((N, D), jnp.float32),
    )(x, W1[:D], W1[D:], scalars, b1.reshape(1, D), W2, b2.reshape(1, D))


def _remap(idx_v, gs, j, lo):
    for k in range(CHUNK // 16):
        sl = pl.ds(k * 16, 16)
        srcv = idx_v[gs, j, 0, sl]
        dl = idx_v[gs, j, 1, sl] - lo
        ok = (dl >= 0) & (dl < HALF)
        neg1 = jnp.full((16,), -1, jnp.int32)
        idx_v[gs, j, 0, sl] = jnp.where(ok, srcv, neg1)
        idx_v[gs, j, 1, sl] = jnp.where(ok, dl, neg1)


def _gather_idx(idx_v, gs, j):
    return plsc.Indices(idx_v.at[gs, j, 0], ignored_value=-1)


def _scatter_idx(idx_v, gs, j):
    return plsc.Indices(idx_v.at[gs, j, 1], ignored_value=-1)


def _sc_body(msg_hbm, eidx_hbm, zeros_hbm, out_hbm, idx_v, bufs, acc,
             semg, semr, semw):
    c = lax.axis_index("c")
    s = lax.axis_index("s")
    lo = c * HALF
    cbase = s * CHUNKS_PER_TILE

    def fetch_group(g, gslot):
        return pltpu.make_async_copy(
            eidx_hbm.at[pl.ds(cbase + g * GB, GB)], idx_v.at[gslot],
            semg[gslot])

    fetch_group(0, 0).start()
    fetch_group(1, 1).start()

    base = s * ROWS_PER_TILE
    pltpu.sync_copy(zeros_hbm, acc.at[pl.ds(base, ROWS_PER_TILE)])
    plsc.subcore_barrier()

    @pl.loop(0, NGRP, step=2)
    def _groups(go):
        for gp in range(2):
            g = go + gp
            fetch_group(g, gp).wait()
            for j in range(GB):
                t = g * GB + j
                b = j % NR

                gw, jw = (gp ^ 1, j + NR) if j < NR else (gp, j - NR)

                @pl.when(t >= NR)
                def _():
                    pltpu.make_async_copy(
                        bufs[b], acc.at[_scatter_idx(idx_v, gw, jw)],
                        semw[b]).wait()

                _remap(idx_v, gp, j, lo)
                pltpu.async_copy(msg_hbm.at[_gather_idx(idx_v, gp, j)],
                                 bufs[b], semr[b])

                bs = (j + 1) % NR
                gs_, js = (gp ^ 1, j + GB - LAG) if j < LAG else (gp, j - LAG)

                @pl.when(t >= LAG)
                def _():
                    pltpu.make_async_copy(
                        msg_hbm.at[_gather_idx(idx_v, gs_, js)],
                        bufs[bs], semr[bs]).wait()
                    pltpu.async_copy(bufs[bs],
                                     acc.at[_scatter_idx(idx_v, gs_, js)],
                                     semw[bs], add=True)

                if j == NR:
                    @pl.when((g >= 1) & (g + 1 < NGRP))
                    def _():
                        fetch_group(g + 1, gp ^ 1).start()

    for t in range(CHUNKS_PER_TILE - LAG, CHUNKS_PER_TILE):
        b = t % NR
        gs_, js = (t // GB) % 2, t % GB
        pltpu.make_async_copy(msg_hbm.at[_gather_idx(idx_v, gs_, js)],
                              bufs[b], semr[b]).wait()
        pltpu.sync_copy(bufs[b], acc.at[_scatter_idx(idx_v, gs_, js)],
                        add=True)
    tl = CHUNKS_PER_TILE - LAG - 1
    pltpu.make_async_copy(bufs[tl % NR],
                          acc.at[_scatter_idx(idx_v, (tl // GB) % 2,
                                              tl % GB)],
                          semw[tl % NR]).wait()
    plsc.subcore_barrier()

    gbase = lo + base

    @pl.when(gbase + ROWS_PER_TILE <= N)
    def _():
        pltpu.sync_copy(acc.at[pl.ds(base, ROWS_PER_TILE)],
                        out_hbm.at[pl.ds(gbase, ROWS_PER_TILE)])

    @pl.when(gbase + ROWS_PER_TILE > N)
    def _():
        pltpu.sync_copy(acc.at[pl.ds(base, LAST_ROWS)],
                        out_hbm.at[pl.ds(gbase, LAST_ROWS)])


def _sc_aggregate(node_msg, eidx, zeros):
    mesh = plsc.VectorSubcoreMesh(core_axis_name="c", subcore_axis_name="s",
                                  num_cores=NC)
    k = pl.kernel(
        _sc_body,
        mesh=mesh,
        out_type=jax.ShapeDtypeStruct((N, D), jnp.float32),
        scratch_types=[
            pltpu.VMEM((2, GB, 2, CHUNK), jnp.int32),
            [pltpu.VMEM((CHUNK, D), jnp.float32)] * NR,
            pltpu.VMEM_SHARED((HALF, D), jnp.float32),
            [pltpu.SemaphoreType.DMA] * 2,
            [pltpu.SemaphoreType.DMA] * NR,
            [pltpu.SemaphoreType.DMA] * NR,
        ],
    )
    return k(node_msg, eidx, zeros)


def kernel(x, scalars, edge_index, W1, b1, W2, b2):
    node_msg = _node_mlp(x, scalars, W1, b1, W2, b2)
    ei = edge_index.astype(jnp.int32)
    pad = jnp.full((2, CHUNKS_PAD * CHUNK - E), -1, jnp.int32)
    eidx = (jnp.concatenate([ei, pad], axis=1)
            .reshape(2, CHUNKS_PAD, CHUNK).transpose(1, 0, 2))
    zeros = jnp.zeros((ROWS_PER_TILE, D), jnp.float32)
    return _sc_aggregate(node_msg, eidx, zeros)


# speedup vs baseline: 1.0236x; 1.0236x over previous
import functools

import jax
import jax.numpy as jnp
from jax import lax
from jax.experimental import pallas as pl
from jax.experimental.pallas import tpu as pltpu
from jax.experimental.pallas import tpu_sc as plsc

N = 10000
E = 320000
D = 128
NC = 2
NS = 16
CHUNK = 128
NR = 5
NI = 2 * NR
LAG = NR - 1
TOTAL_CHUNKS = E // CHUNK
CHUNKS_PER_TILE = 160
HALF = 5120
ROWS_PER_TILE = HALF // NS
LAST_ROWS = N - (NC * HALF - ROWS_PER_TILE)


def _mlp_body(x_ref, w1a_ref, w1b_ref, s_ref, b1_ref, w2_ref, b2_ref,
              o_ref, z_ref):
    b1eff = b1_ref[...] + jnp.dot(s_ref[...], w1b_ref[...],
                                  preferred_element_type=jnp.float32)
    h = jnp.dot(x_ref[...], w1a_ref[...], preferred_element_type=jnp.float32)
    h = jnp.maximum(h + b1eff, 0.0)
    o = jnp.dot(h, w2_ref[...], preferred_element_type=jnp.float32)
    o_ref[...] = o + b2_ref[...]
    z_ref[...] = jnp.zeros_like(z_ref)


def _node_mlp(x, scalars, W1, b1, W2, b2):
    blk = 1000
    grid = N // blk
    zblk = ROWS_PER_TILE // grid
    full = lambda shape: pl.BlockSpec(shape, lambda i: (0,) * len(shape))
    return pl.pallas_call(
        _mlp_body,
        grid=(grid,),
        in_specs=[
            pl.BlockSpec((blk, D), lambda i: (i, 0)),
            full((D, D)),
            full((4, D)),
            full((1, 4)),
            full((1, D)),
            full((D, D)),
            full((1, D)),
        ],
        out_specs=[pl.BlockSpec((blk, D), lambda i: (i, 0)),
                   pl.BlockSpec((zblk, D), lambda i: (i, 0))],
        out_shape=[jax.ShapeDtypeStruct((N, D), jnp.float32),
                   jax.ShapeDtypeStruct((ROWS_PER_TILE, D), jnp.float32)],
    )(x, W1[:D], W1[D:], scalars, b1.reshape(1, D), W2, b2.reshape(1, D))


def _gid(s, chunk):
    return s + NS * chunk


def _idx_copies(eidx_hbm, s, chunk, idx_v, q, sem_s, sem_d):
    off = jnp.minimum(_gid(s, chunk), TOTAL_CHUNKS - 1) * CHUNK
    return (
        pltpu.make_async_copy(eidx_hbm.at[0, pl.ds(off, CHUNK)],
                              idx_v.at[q, 0], sem_s),
        pltpu.make_async_copy(eidx_hbm.at[1, pl.ds(off, CHUNK)],
                              idx_v.at[q, 1], sem_d),
    )


def _remap(idx_v, q, lo, lim):
    for j in range(CHUNK // 16):
        sl = pl.ds(j * 16, 16)
        srcv = idx_v[q, 0, sl]
        dl = idx_v[q, 1, sl] - lo
        ok = (dl >= 0) & (dl < lim)
        neg1 = jnp.full((16,), -1, jnp.int32)
        idx_v[q, 0, sl] = jnp.where(ok, srcv, neg1)
        idx_v[q, 1, sl] = jnp.where(ok, dl, neg1)


def _gather_idx(idx_v, q):
    return plsc.Indices(idx_v.at[q, 0], ignored_value=-1)


def _scatter_idx(idx_v, q):
    return plsc.Indices(idx_v.at[q, 1], ignored_value=-1)


def _sc_body(msg_hbm, eidx_hbm, zeros_hbm, out_hbm, idx_v, bufs, acc,
             semi, semi2, semr, semw):
    c = lax.axis_index("c")
    s = lax.axis_index("s")
    lo = c * HALF

    for q in range(NR):
        for d in _idx_copies(eidx_hbm, s, q, idx_v, q, semi[q], semi2[q]):
            d.start()

    base = s * ROWS_PER_TILE
    pltpu.sync_copy(zeros_hbm, acc.at[pl.ds(base, ROWS_PER_TILE)])
    plsc.subcore_barrier()

    @pl.loop(0, CHUNKS_PER_TILE, step=NI)
    def _edges(g):
        for i in range(NI):
            chunk = g + i
            b = i % NR

            @pl.when(chunk >= NR)
            def _():
                pltpu.make_async_copy(
                    bufs[b], acc.at[_scatter_idx(idx_v, (i + NR) % NI)],
                    semw[b]).wait()

            for d in _idx_copies(eidx_hbm, s, chunk, idx_v, i,
                                 semi[i], semi2[i]):
                d.wait()
            lim = jnp.where(_gid(s, chunk) < TOTAL_CHUNKS, HALF, 0)
            _remap(idx_v, i, lo, lim)
            pltpu.async_copy(msg_hbm.at[_gather_idx(idx_v, i)], bufs[b],
                             semr[b])

            @pl.when(chunk >= LAG)
            def _():
                bs = (i + 1) % NR
                qs = (i + NI - LAG) % NI
                pltpu.make_async_copy(msg_hbm.at[_gather_idx(idx_v, qs)],
                                      bufs[bs], semr[bs]).wait()
                pltpu.async_copy(bufs[bs], acc.at[_scatter_idx(idx_v, qs)],
                                 semw[bs], add=True)

            @pl.when(chunk + NR < CHUNKS_PER_TILE)
            def _():
                q2 = (i + NR) % NI
                for d in _idx_copies(eidx_hbm, s, chunk + NR, idx_v, q2,
                                     semi[q2], semi2[q2]):
                    d.start()

    for t in range(LAG):
        ct = CHUNKS_PER_TILE - LAG + t
        bs = ct % NR
        qs = ct % NI
        pltpu.make_async_copy(msg_hbm.at[_gather_idx(idx_v, qs)],
                              bufs[bs], semr[bs]).wait()
        pltpu.sync_copy(bufs[bs], acc.at[_scatter_idx(idx_v, qs)], add=True)
    cl = CHUNKS_PER_TILE - LAG - 1
    pltpu.make_async_copy(bufs[cl % NR],
                          acc.at[_scatter_idx(idx_v, cl % NI)],
                          semw[cl % NR]).wait()
    plsc.subcore_barrier()

    gbase = lo + base

    @pl.when(gbase + ROWS_PER_TILE <= N)
    def _():
        pltpu.sync_copy(acc.at[pl.ds(base, ROWS_PER_TILE)],
                        out_hbm.at[pl.ds(gbase, ROWS_PER_TILE)])

    @pl.when(gbase + ROWS_PER_TILE > N)
    def _():
        pltpu.sync_copy(acc.at[pl.ds(base, LAST_ROWS)],
                        out_hbm.at[pl.ds(gbase, LAST_ROWS)])


def _sc_aggregate(node_msg, eidx, zeros):
    mesh = plsc.VectorSubcoreMesh(core_axis_name="c", subcore_axis_name="s",
                                  num_cores=NC)
    k = pl.kernel(
        _sc_body,
        mesh=mesh,
        out_type=jax.ShapeDtypeStruct((N, D), jnp.float32),
        scratch_types=[
            pltpu.VMEM((NI, 2, CHUNK), jnp.int32),
            [pltpu.VMEM((CHUNK, D), jnp.float32)] * NR,
            pltpu.VMEM_SHARED((HALF, D), jnp.float32),
            [pltpu.SemaphoreType.DMA] * NI,
            [pltpu.SemaphoreType.DMA] * NI,
            [pltpu.SemaphoreType.DMA] * NR,
            [pltpu.SemaphoreType.DMA] * NR,
        ],
    )
    return k(node_msg, eidx, zeros)


def kernel(x, scalars, edge_index, W1, b1, W2, b2):
    node_msg, zeros = _node_mlp(x, scalars, W1, b1, W2, b2)
    eidx = edge_index.astype(jnp.int32)
    return _sc_aggregate(node_msg, eidx, zeros)


# speedup vs baseline: 1.0424x; 1.0184x over previous
import functools

import jax
import jax.numpy as jnp
from jax import lax
from jax.experimental import pallas as pl
from jax.experimental.pallas import tpu as pltpu
from jax.experimental.pallas import tpu_sc as plsc

N = 10000
E = 320000
D = 128
NC = 2
NS = 16
CHUNK = 128
NR = 5
NI = 2 * NR
LAG = NR - 1
TOTAL_CHUNKS = E // CHUNK
CHUNKS_PER_TILE = 160
HALF = 5120
ROWS_PER_TILE = HALF // NS
LAST_ROWS = N - (NC * HALF - ROWS_PER_TILE)


def _mlp_body(x_ref, w1a_ref, w1b_ref, s_ref, b1_ref, w2_ref, b2_ref,
              o_ref, z_ref):
    b1eff = b1_ref[...] + jnp.dot(s_ref[...], w1b_ref[...],
                                  preferred_element_type=jnp.float32)
    h = jnp.dot(x_ref[...], w1a_ref[...], preferred_element_type=jnp.float32)
    h = jnp.maximum(h + b1eff, 0.0)
    o = jnp.dot(h, w2_ref[...], preferred_element_type=jnp.float32)
    o_ref[...] = o + b2_ref[...]
    z_ref[...] = jnp.zeros_like(z_ref)


def _node_mlp(x, scalars, W1, b1, W2, b2):
    blk = 2000
    grid = N // blk
    zblk = ROWS_PER_TILE // grid
    full = lambda shape: pl.BlockSpec(shape, lambda i: (0,) * len(shape))
    return pl.pallas_call(
        _mlp_body,
        grid=(grid,),
        in_specs=[
            pl.BlockSpec((blk, D), lambda i: (i, 0)),
            full((D, D)),
            full((4, D)),
            full((1, 4)),
            full((1, D)),
            full((D, D)),
            full((1, D)),
        ],
        out_specs=[pl.BlockSpec((blk, D), lambda i: (i, 0)),
                   pl.BlockSpec((zblk, D), lambda i: (i, 0))],
        out_shape=[jax.ShapeDtypeStruct((N, D), jnp.float32),
                   jax.ShapeDtypeStruct((ROWS_PER_TILE, D), jnp.float32)],
    )(x, W1[:D], W1[D:], scalars, b1.reshape(1, D), W2, b2.reshape(1, D))


def _gid(s, chunk):
    return s + NS * chunk


def _idx_copies(eidx_hbm, s, chunk, idx_v, q, sem_s, sem_d):
    off = jnp.minimum(_gid(s, chunk), TOTAL_CHUNKS - 1) * CHUNK
    return (
        pltpu.make_async_copy(eidx_hbm.at[0, pl.ds(off, CHUNK)],
                              idx_v.at[q, 0], sem_s),
        pltpu.make_async_copy(eidx_hbm.at[1, pl.ds(off, CHUNK)],
                              idx_v.at[q, 1], sem_d),
    )


def _remap(idx_v, q, lo, lim):
    for j in range(CHUNK // 16):
        sl = pl.ds(j * 16, 16)
        srcv = idx_v[q, 0, sl]
        dl = idx_v[q, 1, sl] - lo
        ok = (dl >= 0) & (dl < lim)
        neg1 = jnp.full((16,), -1, jnp.int32)
        idx_v[q, 0, sl] = jnp.where(ok, srcv, neg1)
        idx_v[q, 1, sl] = jnp.where(ok, dl, neg1)


def _gather_idx(idx_v, q):
    return plsc.Indices(idx_v.at[q, 0], ignored_value=-1)


def _scatter_idx(idx_v, q):
    return plsc.Indices(idx_v.at[q, 1], ignored_value=-1)


def _sc_body(msg_hbm, eidx_hbm, zeros_hbm, out_hbm, idx_v, bufs, acc,
             semi, semi2, semr, semw):
    c = lax.axis_index("c")
    s = lax.axis_index("s")
    lo = c * HALF

    for q in range(NR):
        for d in _idx_copies(eidx_hbm, s, q, idx_v, q, semi[q], semi2[q]):
            d.start()

    base = s * ROWS_PER_TILE
    pltpu.sync_copy(zeros_hbm, acc.at[pl.ds(base, ROWS_PER_TILE)])
    plsc.subcore_barrier()

    @pl.loop(0, CHUNKS_PER_TILE, step=NI)
    def _edges(g):
        for i in range(NI):
            chunk = g + i
            b = i % NR

            @pl.when(chunk >= NR)
            def _():
                pltpu.make_async_copy(
                    bufs[b], acc.at[_scatter_idx(idx_v, (i + NR) % NI)],
                    semw[b]).wait()

            for d in _idx_copies(eidx_hbm, s, chunk, idx_v, i,
                                 semi[i], semi2[i]):
                d.wait()
            lim = jnp.where(_gid(s, chunk) < TOTAL_CHUNKS, HALF, 0)
            _remap(idx_v, i, lo, lim)
            pltpu.async_copy(msg_hbm.at[_gather_idx(idx_v, i)], bufs[b],
                             semr[b])

            @pl.when(chunk >= LAG)
            def _():
                bs = (i + 1) % NR
                qs = (i + NI - LAG) % NI
                pltpu.make_async_copy(msg_hbm.at[_gather_idx(idx_v, qs)],
                                      bufs[bs], semr[bs]).wait()
                pltpu.async_copy(bufs[bs], acc.at[_scatter_idx(idx_v, qs)],
                                 semw[bs], add=True)

            @pl.when(chunk + NR < CHUNKS_PER_TILE)
            def _():
                q2 = (i + NR) % NI
                for d in _idx_copies(eidx_hbm, s, chunk + NR, idx_v, q2,
                                     semi[q2], semi2[q2]):
                    d.start()

    for t in range(LAG):
        ct = CHUNKS_PER_TILE - LAG + t
        bs = ct % NR
        qs = ct % NI
        pltpu.make_async_copy(msg_hbm.at[_gather_idx(idx_v, qs)],
                              bufs[bs], semr[bs]).wait()
        pltpu.sync_copy(bufs[bs], acc.at[_scatter_idx(idx_v, qs)], add=True)
    cl = CHUNKS_PER_TILE - LAG - 1
    pltpu.make_async_copy(bufs[cl % NR],
                          acc.at[_scatter_idx(idx_v, cl % NI)],
                          semw[cl % NR]).wait()
    plsc.subcore_barrier()

    gbase = lo + base

    @pl.when(gbase + ROWS_PER_TILE <= N)
    def _():
        pltpu.sync_copy(acc.at[pl.ds(base, ROWS_PER_TILE)],
                        out_hbm.at[pl.ds(gbase, ROWS_PER_TILE)])

    @pl.when(gbase + ROWS_PER_TILE > N)
    def _():
        pltpu.sync_copy(acc.at[pl.ds(base, LAST_ROWS)],
                        out_hbm.at[pl.ds(gbase, LAST_ROWS)])


def _sc_aggregate(node_msg, eidx, zeros):
    mesh = plsc.VectorSubcoreMesh(core_axis_name="c", subcore_axis_name="s",
                                  num_cores=NC)
    k = pl.kernel(
        _sc_body,
        mesh=mesh,
        out_type=jax.ShapeDtypeStruct((N, D), jnp.float32),
        scratch_types=[
            pltpu.VMEM((NI, 2, CHUNK), jnp.int32),
            [pltpu.VMEM((CHUNK, D), jnp.float32)] * NR,
            pltpu.VMEM_SHARED((HALF, D), jnp.float32),
            [pltpu.SemaphoreType.DMA] * NI,
            [pltpu.SemaphoreType.DMA] * NI,
            [pltpu.SemaphoreType.DMA] * NR,
            [pltpu.SemaphoreType.DMA] * NR,
        ],
    )
    return k(node_msg, eidx, zeros)


def kernel(x, scalars, edge_index, W1, b1, W2, b2):
    node_msg, zeros = _node_mlp(x, scalars, W1, b1, W2, b2)
    eidx = edge_index.astype(jnp.int32)
    return _sc_aggregate(node_msg, eidx, zeros)


# speedup vs baseline: 1.0512x; 1.0084x over previous
import functools

import jax
import jax.numpy as jnp
from jax import lax
from jax.experimental import pallas as pl
from jax.experimental.pallas import tpu as pltpu
from jax.experimental.pallas import tpu_sc as plsc

N = 10000
E = 320000
D = 128
NC = 2
NS = 16
CHUNK = 128
NR = 5
NI = 2 * NR
LAG = NR - 1
TOTAL_CHUNKS = E // CHUNK
CHUNKS_PER_TILE = 160
HALF = 5120
ROWS_PER_TILE = HALF // NS
LAST_ROWS = N - (NC * HALF - ROWS_PER_TILE)


def _mlp_body(x_ref, w1a_ref, w1b_ref, s_ref, b1_ref, w2_ref, b2_ref,
              o_ref, z_ref):
    b1eff = b1_ref[...] + jnp.dot(s_ref[...], w1b_ref[...],
                                  preferred_element_type=jnp.float32)
    h = jnp.dot(x_ref[...], w1a_ref[...], preferred_element_type=jnp.float32)
    h = jnp.maximum(h + b1eff, 0.0)
    o = jnp.dot(h, w2_ref[...], preferred_element_type=jnp.float32)
    o_ref[...] = o + b2_ref[...]
    z_ref[...] = jnp.zeros_like(z_ref)


def _node_mlp(x, scalars, W1, b1, W2, b2):
    blk = 5000
    grid = N // blk
    zblk = ROWS_PER_TILE // grid
    full = lambda shape: pl.BlockSpec(shape, lambda i: (0,) * len(shape))
    return pl.pallas_call(
        _mlp_body,
        grid=(grid,),
        in_specs=[
            pl.BlockSpec((blk, D), lambda i: (i, 0)),
            full((D, D)),
            full((4, D)),
            full((1, 4)),
            full((1, D)),
            full((D, D)),
            full((1, D)),
        ],
        out_specs=[pl.BlockSpec((blk, D), lambda i: (i, 0)),
                   pl.BlockSpec((zblk, D), lambda i: (i, 0))],
        out_shape=[jax.ShapeDtypeStruct((N, D), jnp.float32),
                   jax.ShapeDtypeStruct((ROWS_PER_TILE, D), jnp.float32)],
    )(x, W1[:D], W1[D:], scalars, b1.reshape(1, D), W2, b2.reshape(1, D))


def _gid(s, chunk):
    return s + NS * chunk


def _idx_copies(eidx_hbm, s, chunk, idx_v, q, sem_s, sem_d):
    off = jnp.minimum(_gid(s, chunk), TOTAL_CHUNKS - 1) * CHUNK
    return (
        pltpu.make_async_copy(eidx_hbm.at[0, pl.ds(off, CHUNK)],
                              idx_v.at[q, 0], sem_s),
        pltpu.make_async_copy(eidx_hbm.at[1, pl.ds(off, CHUNK)],
                              idx_v.at[q, 1], sem_d),
    )


def _remap(idx_v, q, lo, lim):
    for j in range(CHUNK // 16):
        sl = pl.ds(j * 16, 16)
        srcv = idx_v[q, 0, sl]
        dl = idx_v[q, 1, sl] - lo
        ok = (dl >= 0) & (dl < lim)
        neg1 = jnp.full((16,), -1, jnp.int32)
        idx_v[q, 0, sl] = jnp.where(ok, srcv, neg1)
        idx_v[q, 1, sl] = jnp.where(ok, dl, neg1)


def _gather_idx(idx_v, q):
    return plsc.Indices(idx_v.at[q, 0], ignored_value=-1)


def _scatter_idx(idx_v, q):
    return plsc.Indices(idx_v.at[q, 1], ignored_value=-1)


def _sc_body(msg_hbm, eidx_hbm, zeros_hbm, out_hbm, idx_v, bufs, acc,
             semi, semi2, semr, semw):
    c = lax.axis_index("c")
    s = lax.axis_index("s")
    lo = c * HALF

    for q in range(NR):
        for d in _idx_copies(eidx_hbm, s, q, idx_v, q, semi[q], semi2[q]):
            d.start()

    base = s * ROWS_PER_TILE
    pltpu.sync_copy(zeros_hbm, acc.at[pl.ds(base, ROWS_PER_TILE)])
    plsc.subcore_barrier()

    @pl.loop(0, CHUNKS_PER_TILE, step=NI)
    def _edges(g):
        for i in range(NI):
            chunk = g + i
            b = i % NR

            @pl.when(chunk >= NR)
            def _():
                pltpu.make_async_copy(
                    bufs[b], acc.at[_scatter_idx(idx_v, (i + NR) % NI)],
                    semw[b]).wait()

            for d in _idx_copies(eidx_hbm, s, chunk, idx_v, i,
                                 semi[i], semi2[i]):
                d.wait()
            lim = jnp.where(_gid(s, chunk) < TOTAL_CHUNKS, HALF, 0)
            _remap(idx_v, i, lo, lim)
            pltpu.async_copy(msg_hbm.at[_gather_idx(idx_v, i)], bufs[b],
                             semr[b])

            @pl.when(chunk >= LAG)
            def _():
                bs = (i + 1) % NR
                qs = (i + NI - LAG) % NI
                pltpu.make_async_copy(msg_hbm.at[_gather_idx(idx_v, qs)],
                                      bufs[bs], semr[bs]).wait()
                pltpu.async_copy(bufs[bs], acc.at[_scatter_idx(idx_v, qs)],
                                 semw[bs], add=True)

            @pl.when(chunk + NR < CHUNKS_PER_TILE)
            def _():
                q2 = (i + NR) % NI
                for d in _idx_copies(eidx_hbm, s, chunk + NR, idx_v, q2,
                                     semi[q2], semi2[q2]):
                    d.start()

    for t in range(LAG):
        ct = CHUNKS_PER_TILE - LAG + t
        bs = ct % NR
        qs = ct % NI
        pltpu.make_async_copy(msg_hbm.at[_gather_idx(idx_v, qs)],
                              bufs[bs], semr[bs]).wait()
        pltpu.sync_copy(bufs[bs], acc.at[_scatter_idx(idx_v, qs)], add=True)
    cl = CHUNKS_PER_TILE - LAG - 1
    pltpu.make_async_copy(bufs[cl % NR],
                          acc.at[_scatter_idx(idx_v, cl % NI)],
                          semw[cl % NR]).wait()
    plsc.subcore_barrier()

    gbase = lo + base

    @pl.when(gbase + ROWS_PER_TILE <= N)
    def _():
        pltpu.sync_copy(acc.at[pl.ds(base, ROWS_PER_TILE)],
                        out_hbm.at[pl.ds(gbase, ROWS_PER_TILE)])

    @pl.when(gbase + ROWS_PER_TILE > N)
    def _():
        pltpu.sync_copy(acc.at[pl.ds(base, LAST_ROWS)],
                        out_hbm.at[pl.ds(gbase, LAST_ROWS)])


def _sc_aggregate(node_msg, eidx, zeros):
    mesh = plsc.VectorSubcoreMesh(core_axis_name="c", subcore_axis_name="s",
                                  num_cores=NC)
    k = pl.kernel(
        _sc_body,
        mesh=mesh,
        out_type=jax.ShapeDtypeStruct((N, D), jnp.float32),
        scratch_types=[
            pltpu.VMEM((NI, 2, CHUNK), jnp.int32),
            [pltpu.VMEM((CHUNK, D), jnp.float32)] * NR,
            pltpu.VMEM_SHARED((HALF, D), jnp.float32),
            [pltpu.SemaphoreType.DMA] * NI,
            [pltpu.SemaphoreType.DMA] * NI,
            [pltpu.SemaphoreType.DMA] * NR,
            [pltpu.SemaphoreType.DMA] * NR,
        ],
    )
    return k(node_msg, eidx, zeros)


def kernel(x, scalars, edge_index, W1, b1, W2, b2):
    node_msg, zeros = _node_mlp(x, scalars, W1, b1, W2, b2)
    eidx = edge_index.astype(jnp.int32)
    return _sc_aggregate(node_msg, eidx, zeros)


# speedup vs baseline: 1.0518x; 1.0006x over previous
import functools

import jax
import jax.numpy as jnp
from jax import lax
from jax.experimental import pallas as pl
from jax.experimental.pallas import tpu as pltpu
from jax.experimental.pallas import tpu_sc as plsc

N = 10000
E = 320000
D = 128
NC = 2
NS = 16
CHUNK = 128
NR = 5
NI = 2 * NR
LAG = NR - 1
TOTAL_CHUNKS = E // CHUNK
CHUNKS_PER_TILE = 160
HALF = 5120
ROWS_PER_TILE = HALF // NS
LAST_ROWS = N - (NC * HALF - ROWS_PER_TILE)


def _mlp_body(x_ref, w1a_ref, w1b_ref, s_ref, b1_ref, w2_ref, b2_ref,
              o_ref, z_ref):
    b1eff = b1_ref[...] + jnp.dot(s_ref[...], w1b_ref[...],
                                  preferred_element_type=jnp.float32)
    h = jnp.dot(x_ref[...], w1a_ref[...], preferred_element_type=jnp.float32)
    h = jnp.maximum(h + b1eff, 0.0)
    o = jnp.dot(h, w2_ref[...], preferred_element_type=jnp.float32)
    o_ref[...] = o + b2_ref[...]
    z_ref[...] = jnp.zeros_like(z_ref)


def _node_mlp(x, scalars, W1, b1, W2, b2):
    blk = 10000
    grid = N // blk
    zblk = ROWS_PER_TILE // grid
    full = lambda shape: pl.BlockSpec(shape, lambda i: (0,) * len(shape))
    return pl.pallas_call(
        _mlp_body,
        grid=(grid,),
        in_specs=[
            pl.BlockSpec((blk, D), lambda i: (i, 0)),
            full((D, D)),
            full((4, D)),
            full((1, 4)),
            full((1, D)),
            full((D, D)),
            full((1, D)),
        ],
        out_specs=[pl.BlockSpec((blk, D), lambda i: (i, 0)),
                   pl.BlockSpec((zblk, D), lambda i: (i, 0))],
        out_shape=[jax.ShapeDtypeStruct((N, D), jnp.float32),
                   jax.ShapeDtypeStruct((ROWS_PER_TILE, D), jnp.float32)],
    )(x, W1[:D], W1[D:], scalars, b1.reshape(1, D), W2, b2.reshape(1, D))


def _gid(s, chunk):
    return s + NS * chunk


def _idx_copies(eidx_hbm, s, chunk, idx_v, q, sem_s, sem_d):
    off = jnp.minimum(_gid(s, chunk), TOTAL_CHUNKS - 1) * CHUNK
    return (
        pltpu.make_async_copy(eidx_hbm.at[0, pl.ds(off, CHUNK)],
                              idx_v.at[q, 0], sem_s),
        pltpu.make_async_copy(eidx_hbm.at[1, pl.ds(off, CHUNK)],
                              idx_v.at[q, 1], sem_d),
    )


def _remap(idx_v, q, lo, lim):
    for j in range(CHUNK // 16):
        sl = pl.ds(j * 16, 16)
        srcv = idx_v[q, 0, sl]
        dl = idx_v[q, 1, sl] - lo
        ok = (dl >= 0) & (dl < lim)
        neg1 = jnp.full((16,), -1, jnp.int32)
        idx_v[q, 0, sl] = jnp.where(ok, srcv, neg1)
        idx_v[q, 1, sl] = jnp.where(ok, dl, neg1)


def _gather_idx(idx_v, q):
    return plsc.Indices(idx_v.at[q, 0], ignored_value=-1)


def _scatter_idx(idx_v, q):
    return plsc.Indices(idx_v.at[q, 1], ignored_value=-1)


def _sc_body(msg_hbm, eidx_hbm, zeros_hbm, out_hbm, idx_v, bufs, acc,
             semi, semi2, semr, semw):
    c = lax.axis_index("c")
    s = lax.axis_index("s")
    lo = c * HALF

    for q in range(NR):
        for d in _idx_copies(eidx_hbm, s, q, idx_v, q, semi[q], semi2[q]):
            d.start()

    base = s * ROWS_PER_TILE
    pltpu.sync_copy(zeros_hbm, acc.at[pl.ds(base, ROWS_PER_TILE)])
    plsc.subcore_barrier()

    @pl.loop(0, CHUNKS_PER_TILE, step=NI)
    def _edges(g):
        for i in range(NI):
            chunk = g + i
            b = i % NR

            @pl.when(chunk >= NR)
            def _():
                pltpu.make_async_copy(
                    bufs[b], acc.at[_scatter_idx(idx_v, (i + NR) % NI)],
                    semw[b]).wait()

            for d in _idx_copies(eidx_hbm, s, chunk, idx_v, i,
                                 semi[i], semi2[i]):
                d.wait()
            lim = jnp.where(_gid(s, chunk) < TOTAL_CHUNKS, HALF, 0)
            _remap(idx_v, i, lo, lim)
            pltpu.async_copy(msg_hbm.at[_gather_idx(idx_v, i)], bufs[b],
                             semr[b])

            @pl.when(chunk >= LAG)
            def _():
                bs = (i + 1) % NR
                qs = (i + NI - LAG) % NI
                pltpu.make_async_copy(msg_hbm.at[_gather_idx(idx_v, qs)],
                                      bufs[bs], semr[bs]).wait()
                pltpu.async_copy(bufs[bs], acc.at[_scatter_idx(idx_v, qs)],
                                 semw[bs], add=True)

            @pl.when(chunk + NR < CHUNKS_PER_TILE)
            def _():
                q2 = (i + NR) % NI
                for d in _idx_copies(eidx_hbm, s, chunk + NR, idx_v, q2,
                                     semi[q2], semi2[q2]):
                    d.start()

    for t in range(LAG):
        ct = CHUNKS_PER_TILE - LAG + t
        bs = ct % NR
        qs = ct % NI
        pltpu.make_async_copy(msg_hbm.at[_gather_idx(idx_v, qs)],
                              bufs[bs], semr[bs]).wait()
        pltpu.sync_copy(bufs[bs], acc.at[_scatter_idx(idx_v, qs)], add=True)
    cl = CHUNKS_PER_TILE - LAG - 1
    pltpu.make_async_copy(bufs[cl % NR],
                          acc.at[_scatter_idx(idx_v, cl % NI)],
                          semw[cl % NR]).wait()
    plsc.subcore_barrier()

    gbase = lo + base

    @pl.when(gbase + ROWS_PER_TILE <= N)
    def _():
        pltpu.sync_copy(acc.at[pl.ds(base, ROWS_PER_TILE)],
                        out_hbm.at[pl.ds(gbase, ROWS_PER_TILE)])

    @pl.when(gbase + ROWS_PER_TILE > N)
    def _():
        pltpu.sync_copy(acc.at[pl.ds(base, LAST_ROWS)],
                        out_hbm.at[pl.ds(gbase, LAST_ROWS)])


def _sc_aggregate(node_msg, eidx, zeros):
    mesh = plsc.VectorSubcoreMesh(core_axis_name="c", subcore_axis_name="s",
                                  num_cores=NC)
    k = pl.kernel(
        _sc_body,
        mesh=mesh,
        out_type=jax.ShapeDtypeStruct((N, D), jnp.float32),
        scratch_types=[
            pltpu.VMEM((NI, 2, CHUNK), jnp.int32),
            [pltpu.VMEM((CHUNK, D), jnp.float32)] * NR,
            pltpu.VMEM_SHARED((HALF, D), jnp.float32),
            [pltpu.SemaphoreType.DMA] * NI,
            [pltpu.SemaphoreType.DMA] * NI,
            [pltpu.SemaphoreType.DMA] * NR,
            [pltpu.SemaphoreType.DMA] * NR,
        ],
    )
    return k(node_msg, eidx, zeros)


def kernel(x, scalars, edge_index, W1, b1, W2, b2):
    node_msg, zeros = _node_mlp(x, scalars, W1, b1, W2, b2)
    eidx = edge_index.astype(jnp.int32)
    return _sc_aggregate(node_msg, eidx, zeros)
